# Initial kernel scaffold; baseline (speedup 1.0000x reference)
import jax
import jax.numpy as jnp
from jax.experimental import pallas as pl


def kernel(rgb, d):
    raise NotImplementedError("write your pallas kernel here")


# speedup vs baseline: 79.3555x; 79.3555x over previous
import functools

import jax
import jax.numpy as jnp
from jax import lax
from jax.experimental import pallas as pl
from jax.experimental.pallas import tpu as pltpu
from jax.experimental.pallas import tpu_sc as plsc

_NBINS = 10
_PCT = 0.01
_NB1 = 2048
_NB2 = 2048
_NB3 = 1024
_ROWS = _NBINS + 1


def _p0_body(rgb_ref, d_ref, gray_ref, mn_ref, mx_ref):
    i = pl.program_id(0)
    r = rgb_ref[0, 0]
    g = rgb_ref[0, 1]
    b = rgb_ref[0, 2]
    gray_ref[0, 0] = 0.299 * r + 0.587 * g + 0.114 * b
    dv = d_ref[0, 0]
    mn = jnp.min(dv)
    mx = jnp.max(dv)

    @pl.when(i == 0)
    def _():
        mn_ref[...] = jnp.full((8, 128), jnp.inf, jnp.float32)
        mx_ref[...] = jnp.full((8, 128), -jnp.inf, jnp.float32)

    mn_ref[...] = jnp.minimum(mn_ref[...], mn)
    mx_ref[...] = jnp.maximum(mx_ref[...], mx)


def _tc_gray_minmax(rgb, d):
    B, C, H, W = rgb.shape
    return pl.pallas_call(
        _p0_body,
        grid=(B,),
        in_specs=[
            pl.BlockSpec((1, 3, H, W), lambda i: (i, 0, 0, 0)),
            pl.BlockSpec((1, 1, H, W), lambda i: (i, 0, 0, 0)),
        ],
        out_specs=[
            pl.BlockSpec((1, 1, H, W), lambda i: (i, 0, 0, 0)),
            pl.BlockSpec((8, 128), lambda i: (0, 0)),
            pl.BlockSpec((8, 128), lambda i: (0, 0)),
        ],
        out_shape=[
            jax.ShapeDtypeStruct((B, 1, H, W), jnp.float32),
            jax.ShapeDtypeStruct((8, 128), jnp.float32),
            jax.ShapeDtypeStruct((8, 128), jnp.float32),
        ],
    )(rgb, d)


def _p1_body(dmm_ref, d_ref, bin_ref):
    dmin = dmm_ref[0]
    dmax = dmm_ref[1]
    drange = dmax - dmin
    dv = d_ref[0, 0]
    binv = jnp.zeros(dv.shape, jnp.int32)
    for i in range(1, _NBINS + 1):
        lo = dmin + (jnp.float32(i) * drange) / jnp.float32(_NBINS)
        binv = binv + (dv >= lo).astype(jnp.int32)
    bin_ref[0, 0] = binv


def _tc_bins(d, dmm):
    B, _, H, W = d.shape
    return pl.pallas_call(
        _p1_body,
        grid=(B,),
        in_specs=[
            pl.BlockSpec(memory_space=pltpu.SMEM),
            pl.BlockSpec((1, 1, H, W), lambda i: (i, 0, 0, 0)),
        ],
        out_specs=pl.BlockSpec((1, 1, H, W), lambda i: (i, 0, 0, 0)),
        out_shape=jax.ShapeDtypeStruct((B, 1, H, W), jnp.int32),
    )(dmm, d)


_CHUNK = 4096


def _sc_hist(gray_flat, bin_flat, level, tbl=None):
    n = gray_flat.shape[0]
    info = plsc.get_sparse_core_info()
    nc, ns = info.num_cores, info.num_subcores
    nw = nc * ns
    ew = n // nw
    nch = ew // _CHUNK
    nb = {1: _NB1, 2: _NB2, 3: _NB3}[level]

    mesh = plsc.VectorSubcoreMesh(
        core_axis_name="c", subcore_axis_name="s",
        num_cores=nc, num_subcores=ns)

    scratch = [
        pltpu.VMEM((_CHUNK,), jnp.float32),
        pltpu.VMEM((_CHUNK,), jnp.int32),
        pltpu.VMEM((_ROWS * nb,), jnp.int32),
    ]
    if level > 1:
        scratch.append(pltpu.VMEM((16,), jnp.int32))

    def body(*refs):
        if level == 1:
            gray_hbm, bin_hbm, out_hbm, gbuf, bbuf, hist = refs
            tblv = None
        else:
            gray_hbm, bin_hbm, tbl_hbm, out_hbm, gbuf, bbuf, hist, tblv = refs
        wid = lax.axis_index("s") * nc + lax.axis_index("c")
        base = wid * ew

        def zrow(j, _):
            hist[pl.ds(j * 16, 16)] = jnp.zeros((16,), jnp.int32)
            return 0
        lax.fori_loop(0, (_ROWS * nb) // 16, zrow, 0)

        if level > 1:
            pltpu.sync_copy(tbl_hbm.at[pl.ds(0, 16)], tblv)

        ones = jnp.ones((16,), jnp.int32)

        def chunk(c, _):
            off = base + c * _CHUNK
            pltpu.sync_copy(gray_hbm.at[pl.ds(off, _CHUNK)], gbuf)
            pltpu.sync_copy(bin_hbm.at[pl.ds(off, _CHUNK)], bbuf)

            def vec(j, _):
                s = pl.ds(j * 16, 16)
                bits = lax.bitcast_convert_type(gbuf[s], jnp.int32)
                binv = bbuf[s]
                if level == 1:
                    k = lax.shift_right_logical(bits, 21)
                    plsc.addupdate_scatter(hist, [binv * nb + k], ones)
                elif level == 2:
                    t = plsc.load_gather(tblv, [binv])
                    m = lax.shift_right_logical(bits, 21) == t
                    k = lax.shift_right_logical(bits, 10) & 0x7FF
                    plsc.addupdate_scatter(hist, [binv * nb + k], ones,
                                           mask=m)
                else:
                    t = plsc.load_gather(tblv, [binv])
                    m = lax.shift_right_logical(bits, 10) == t
                    k = bits & 0x3FF
                    plsc.addupdate_scatter(hist, [binv * nb + k], ones,
                                           mask=m)
                return 0

            lax.fori_loop(0, _CHUNK // 16, vec, 0)
            return 0

        lax.fori_loop(0, nch, chunk, 0)
        pltpu.sync_copy(hist, out_hbm.at[wid])

    kern = pl.kernel(
        body,
        out_type=jax.ShapeDtypeStruct((nw, _ROWS * nb), jnp.int32),
        mesh=mesh,
        scratch_types=scratch,
        compiler_params=pltpu.CompilerParams(needs_layout_passes=False),
    )
    if level == 1:
        out = kern(gray_flat, bin_flat)
    else:
        out = kern(gray_flat, bin_flat, tbl)
    return out.reshape(nw, _ROWS, nb)


def _cumsum_rows(h):
    nb = h.shape[1]
    cum = h
    s = 1
    while s < nb:
        z = jnp.zeros((h.shape[0], s), jnp.int32)
        cum = cum + jnp.concatenate([z, cum[:, : nb - s]], axis=1)
        s *= 2
    return cum


def _pick_bucket(h, rank):
    nb = h.shape[1]
    cum = _cumsum_rows(h)
    bucket = jnp.sum((cum <= rank).astype(jnp.int32), axis=1, keepdims=True)
    bucket = jnp.minimum(bucket, nb - 1)
    col = lax.broadcasted_iota(jnp.int32, h.shape, 1)
    below = jnp.sum(jnp.where(col < bucket, h, 0), axis=1, keepdims=True)
    return bucket, rank - below


def _to_row(v, fill):
    rid = lax.broadcasted_iota(jnp.int32, (_ROWS, 128), 0)
    cid = lax.broadcasted_iota(jnp.int32, (_ROWS, 128), 1)
    mat = jnp.where(rid == cid, jnp.broadcast_to(v, (_ROWS, 128)),
                    jnp.zeros((_ROWS, 128), v.dtype))
    row = jnp.sum(mat, axis=0, keepdims=True)
    c = lax.broadcasted_iota(jnp.int32, (1, 128), 1)
    return jnp.where(c >= _NBINS, jnp.asarray(fill, v.dtype), row)


def _from_row(row):
    rid = lax.broadcasted_iota(jnp.int32, (_ROWS, 128), 0)
    cid = lax.broadcasted_iota(jnp.int32, (_ROWS, 128), 1)
    mat = jnp.where(rid == cid, jnp.broadcast_to(row, (_ROWS, 128)),
                    jnp.zeros((_ROWS, 128), jnp.int32))
    return jnp.sum(mat, axis=1, keepdims=True)


def _sum_parts(parts_ref):
    h = parts_ref[0]
    for w in range(1, parts_ref.shape[0]):
        h = h + parts_ref[w]
    return h


def _s1_body(parts_ref, g1_ref, r1_ref):
    h = _sum_parts(parts_ref)
    num = jnp.sum(h, axis=1, keepdims=True)
    kf = jnp.ceil(num.astype(jnp.float32) * jnp.float32(_PCT))
    k = jnp.maximum(kf.astype(jnp.int32) - 1, 0)
    bucket, resid = _pick_bucket(h, k)
    g1_ref[...] = _to_row(bucket, -1)
    r1_ref[...] = _to_row(resid, 0)


def _s23_body(level, parts_ref, grow_ref, rrow_ref, gout_ref, rout_ref):
    h = _sum_parts(parts_ref)
    gprev = _from_row(grow_ref[...])
    rank = _from_row(rrow_ref[...])
    bucket, resid = _pick_bucket(h, rank)
    if level == 2:
        gout_ref[...] = _to_row(gprev * _NB2 + bucket, -1)
        rout_ref[...] = _to_row(resid, 0)
    else:
        tbits = gprev * _NB3 + bucket
        t = lax.bitcast_convert_type(tbits, jnp.float32)
        gout_ref[...] = _to_row(t, -1.0)
        rout_ref[...] = _to_row(resid, 0)


def _tc_select(level, parts, grow=None, rrow=None):
    outs = [
        jax.ShapeDtypeStruct((1, 128),
                             jnp.float32 if level == 3 else jnp.int32),
        jax.ShapeDtypeStruct((1, 128), jnp.int32),
    ]
    if level == 1:
        return pl.pallas_call(_s1_body, out_shape=outs)(parts)
    body = functools.partial(_s23_body, level)
    return pl.pallas_call(body, out_shape=outs)(parts, grow, rrow)


def _p7_body(nsteps, inv_n, trow_ref, gray_ref, bin_ref, dcp_ref, loss_ref,
             acc_ref):
    i = pl.program_id(0)
    g = gray_ref[...]
    binv = bin_ref[...]
    tpix = jnp.full(g.shape, -1.0, jnp.float32)
    for b in range(_ROWS):
        tpix = jnp.where(binv == b, trow_ref[0, b], tpix)
    dcp = jnp.where(g <= tpix, g, 0.0)
    dcp_ref[...] = dcp

    @pl.when(i == 0)
    def _():
        acc_ref[0] = 0.0

    acc_ref[0] = acc_ref[0] + jnp.sum(jnp.abs(dcp))

    @pl.when(i == nsteps - 1)
    def _():
        loss_ref[...] = jnp.full((8, 128), acc_ref[0] * inv_n, jnp.float32)


def _tc_final(gray2d, bin2d, trow):
    rows, cols = gray2d.shape
    blk = 256
    nsteps = rows // blk
    n = rows * cols
    body = functools.partial(_p7_body, nsteps, 1.0 / n)
    return pl.pallas_call(
        body,
        grid=(nsteps,),
        in_specs=[
            pl.BlockSpec(memory_space=pltpu.SMEM),
            pl.BlockSpec((blk, cols), lambda i: (i, 0)),
            pl.BlockSpec((blk, cols), lambda i: (i, 0)),
        ],
        out_specs=[
            pl.BlockSpec((blk, cols), lambda i: (i, 0)),
            pl.BlockSpec((8, 128), lambda i: (0, 0)),
        ],
        out_shape=[
            jax.ShapeDtypeStruct((rows, cols), jnp.float32),
            jax.ShapeDtypeStruct((8, 128), jnp.float32),
        ],
        scratch_shapes=[pltpu.SMEM((1,), jnp.float32)],
    )(trow, gray2d, bin2d)


def kernel(rgb, d):
    B, C, H, W = rgb.shape
    n = B * H * W

    gray, mn_a, mx_a = _tc_gray_minmax(rgb, d)
    dmm = jnp.stack([mn_a[0, 0], mx_a[0, 0]])
    binb = _tc_bins(d, dmm)

    gray_flat = gray.reshape(n)
    bin_flat = binb.reshape(n)

    parts1 = _sc_hist(gray_flat, bin_flat, 1)
    g1, r1 = _tc_select(1, parts1)
    parts2 = _sc_hist(gray_flat, bin_flat, 2, g1.reshape(128))
    g2, r2 = _tc_select(2, parts2, g1, r1)
    parts3 = _sc_hist(gray_flat, bin_flat, 3, g2.reshape(128))
    trow, _ = _tc_select(3, parts3, g2, r2)

    dcp2d, loss_a = _tc_final(gray_flat.reshape(2048, n // 2048),
                              bin_flat.reshape(2048, n // 2048), trow)
    return (loss_a[0, 0], dcp2d.reshape(B, 1, H, W))


# speedup vs baseline: 110.6351x; 1.3942x over previous
import functools

import jax
import jax.numpy as jnp
from jax import lax
from jax.experimental import pallas as pl
from jax.experimental.pallas import tpu as pltpu
from jax.experimental.pallas import tpu_sc as plsc

_NBINS = 10
_PCT = 0.01
_NB1 = 2048
_NB2 = 2048
_NB3 = 1024
_ROWS = _NBINS + 1


def _p0_body(rgb_ref, d_ref, gray_ref, mn_ref, mx_ref):
    i = pl.program_id(0)
    r = rgb_ref[0, 0]
    g = rgb_ref[0, 1]
    b = rgb_ref[0, 2]
    gray_ref[0, 0] = 0.299 * r + 0.587 * g + 0.114 * b
    dv = d_ref[0, 0]
    mn = jnp.min(dv)
    mx = jnp.max(dv)

    @pl.when(i == 0)
    def _():
        mn_ref[...] = jnp.full((8, 128), jnp.inf, jnp.float32)
        mx_ref[...] = jnp.full((8, 128), -jnp.inf, jnp.float32)

    mn_ref[...] = jnp.minimum(mn_ref[...], mn)
    mx_ref[...] = jnp.maximum(mx_ref[...], mx)


def _tc_gray_minmax(rgb, d):
    B, C, H, W = rgb.shape
    return pl.pallas_call(
        _p0_body,
        grid=(B,),
        in_specs=[
            pl.BlockSpec((1, 3, H, W), lambda i: (i, 0, 0, 0)),
            pl.BlockSpec((1, 1, H, W), lambda i: (i, 0, 0, 0)),
        ],
        out_specs=[
            pl.BlockSpec((1, 1, H, W), lambda i: (i, 0, 0, 0)),
            pl.BlockSpec((8, 128), lambda i: (0, 0)),
            pl.BlockSpec((8, 128), lambda i: (0, 0)),
        ],
        out_shape=[
            jax.ShapeDtypeStruct((B, 1, H, W), jnp.float32),
            jax.ShapeDtypeStruct((8, 128), jnp.float32),
            jax.ShapeDtypeStruct((8, 128), jnp.float32),
        ],
    )(rgb, d)


def _p1_body(dmm_ref, d_ref, bin_ref):
    dmin = dmm_ref[0]
    dmax = dmm_ref[1]
    drange = dmax - dmin
    dv = d_ref[0, 0]
    binv = jnp.zeros(dv.shape, jnp.int32)
    for i in range(1, _NBINS + 1):
        lo = dmin + (jnp.float32(i) * drange) / jnp.float32(_NBINS)
        binv = binv + (dv >= lo).astype(jnp.int32)
    bin_ref[0, 0] = binv


def _tc_bins(d, dmm):
    B, _, H, W = d.shape
    return pl.pallas_call(
        _p1_body,
        grid=(B,),
        in_specs=[
            pl.BlockSpec(memory_space=pltpu.SMEM),
            pl.BlockSpec((1, 1, H, W), lambda i: (i, 0, 0, 0)),
        ],
        out_specs=pl.BlockSpec((1, 1, H, W), lambda i: (i, 0, 0, 0)),
        out_shape=jax.ShapeDtypeStruct((B, 1, H, W), jnp.int32),
    )(dmm, d)


_CHUNK = 8192
_UNROLL = 8


def _sc_hist(gray_flat, bin_flat, level, tbl=None):
    n = gray_flat.shape[0]
    info = plsc.get_sparse_core_info()
    nc, ns = info.num_cores, info.num_subcores
    nw = nc * ns
    ew = n // nw
    nch = ew // _CHUNK
    nb = {1: _NB1, 2: _NB2, 3: _NB3}[level]

    mesh = plsc.VectorSubcoreMesh(
        core_axis_name="c", subcore_axis_name="s",
        num_cores=nc, num_subcores=ns)

    scratch = [
        pltpu.VMEM((_CHUNK,), jnp.float32),
        pltpu.VMEM((_CHUNK,), jnp.float32),
        pltpu.VMEM((_CHUNK,), jnp.int32),
        pltpu.VMEM((_CHUNK,), jnp.int32),
        pltpu.VMEM((_ROWS * nb,), jnp.int32),
        pltpu.SemaphoreType.DMA,
        pltpu.SemaphoreType.DMA,
        pltpu.SemaphoreType.DMA,
        pltpu.SemaphoreType.DMA,
    ]
    if level > 1:
        scratch.append(pltpu.VMEM((16,), jnp.int32))

    def body(*refs):
        if level == 1:
            (gray_hbm, bin_hbm, out_hbm, gbuf0, gbuf1, bbuf0, bbuf1, hist,
             sg0, sg1, sb0, sb1) = refs
            tblv = None
        else:
            (gray_hbm, bin_hbm, tbl_hbm, out_hbm, gbuf0, gbuf1, bbuf0, bbuf1,
             hist, sg0, sg1, sb0, sb1, tblv) = refs
        wid = lax.axis_index("s") * nc + lax.axis_index("c")
        base = wid * ew
        gbufs = (gbuf0, gbuf1)
        bbufs = (bbuf0, bbuf1)
        sems = ((sg0, sb0), (sg1, sb1))

        def zrow(j, _):
            for u in range(_UNROLL):
                hist[pl.ds((j * _UNROLL + u) * 16, 16)] = (
                    jnp.zeros((16,), jnp.int32))
            return 0
        lax.fori_loop(0, (_ROWS * nb) // (16 * _UNROLL), zrow, 0)

        if level > 1:
            pltpu.sync_copy(tbl_hbm.at[pl.ds(0, 16)], tblv)

        ones = jnp.ones((16,), jnp.int32)

        def start(c, slot):
            off = base + c * _CHUNK
            pltpu.async_copy(gray_hbm.at[pl.ds(off, _CHUNK)],
                             gbufs[slot], sems[slot][0])
            pltpu.async_copy(bin_hbm.at[pl.ds(off, _CHUNK)],
                             bbufs[slot], sems[slot][1])

        def wait(slot):
            pltpu.make_async_copy(gray_hbm.at[pl.ds(0, _CHUNK)],
                                  gbufs[slot], sems[slot][0]).wait()
            pltpu.make_async_copy(bin_hbm.at[pl.ds(0, _CHUNK)],
                                  bbufs[slot], sems[slot][1]).wait()

        start(0, 0)
        for c in range(nch):
            slot = c & 1
            if c + 1 < nch:
                start(c + 1, 1 - slot)
            wait(slot)
            gb = gbufs[slot]
            bb = bbufs[slot]

            def vec(j, _, gb=gb, bb=bb):
                for u in range(_UNROLL):
                    s = pl.ds(j * (16 * _UNROLL) + u * 16, 16)
                    bits = lax.bitcast_convert_type(gb[s], jnp.int32)
                    binv = bb[s]
                    if level == 1:
                        k = lax.shift_right_logical(bits, 21)
                        plsc.addupdate_scatter(hist, [binv * nb + k], ones)
                    elif level == 2:
                        t = plsc.load_gather(tblv, [binv])
                        m = lax.shift_right_logical(bits, 21) == t
                        k = lax.shift_right_logical(bits, 10) & 0x7FF
                        plsc.addupdate_scatter(hist, [binv * nb + k], ones,
                                               mask=m)
                    else:
                        t = plsc.load_gather(tblv, [binv])
                        m = lax.shift_right_logical(bits, 10) == t
                        k = bits & 0x3FF
                        plsc.addupdate_scatter(hist, [binv * nb + k], ones,
                                               mask=m)
                return 0

            lax.fori_loop(0, _CHUNK // (16 * _UNROLL), vec, 0)

        for r in range(_ROWS):
            pltpu.sync_copy(hist.at[pl.ds(r * nb, nb)],
                            out_hbm.at[wid * 16 + r])

    kern = pl.kernel(
        body,
        out_type=jax.ShapeDtypeStruct((nw * 16, nb), jnp.int32),
        mesh=mesh,
        scratch_types=scratch,
        compiler_params=pltpu.CompilerParams(needs_layout_passes=False),
    )
    if level == 1:
        return kern(gray_flat, bin_flat)
    return kern(gray_flat, bin_flat, tbl)


def _cumsum_rows(h):
    nb = h.shape[1]
    cum = h
    s = 1
    while s < nb:
        z = jnp.zeros((h.shape[0], s), jnp.int32)
        cum = cum + jnp.concatenate([z, cum[:, : nb - s]], axis=1)
        s *= 2
    return cum


def _pick_bucket(h, rank):
    nb = h.shape[1]
    cum = _cumsum_rows(h)
    bucket = jnp.sum((cum <= rank).astype(jnp.int32), axis=1, keepdims=True)
    bucket = jnp.minimum(bucket, nb - 1)
    col = lax.broadcasted_iota(jnp.int32, h.shape, 1)
    below = jnp.sum(jnp.where(col < bucket, h, 0), axis=1, keepdims=True)
    return bucket, rank - below


def _to_row(v, fill):
    rid = lax.broadcasted_iota(jnp.int32, (_ROWS, 128), 0)
    cid = lax.broadcasted_iota(jnp.int32, (_ROWS, 128), 1)
    mat = jnp.where(rid == cid, jnp.broadcast_to(v, (_ROWS, 128)),
                    jnp.zeros((_ROWS, 128), v.dtype))
    row = jnp.sum(mat, axis=0, keepdims=True)
    c = lax.broadcasted_iota(jnp.int32, (1, 128), 1)
    return jnp.where(c >= _NBINS, jnp.asarray(fill, v.dtype), row)


def _from_row(row):
    rid = lax.broadcasted_iota(jnp.int32, (_ROWS, 128), 0)
    cid = lax.broadcasted_iota(jnp.int32, (_ROWS, 128), 1)
    mat = jnp.where(rid == cid, jnp.broadcast_to(row, (_ROWS, 128)),
                    jnp.zeros((_ROWS, 128), jnp.int32))
    return jnp.sum(mat, axis=1, keepdims=True)


def _sum_parts(parts_ref):
    nworkers = parts_ref.shape[0] // 16
    h = parts_ref[0:_ROWS, :]
    for w in range(1, nworkers):
        h = h + parts_ref[w * 16:w * 16 + _ROWS, :]
    return h


def _s1_body(parts_ref, g1_ref, r1_ref):
    h = _sum_parts(parts_ref)
    num = jnp.sum(h, axis=1, keepdims=True)
    kf = jnp.ceil(num.astype(jnp.float32) * jnp.float32(_PCT))
    k = jnp.maximum(kf.astype(jnp.int32) - 1, 0)
    bucket, resid = _pick_bucket(h, k)
    g1_ref[...] = _to_row(bucket, -1)
    r1_ref[...] = _to_row(resid, 0)


def _s23_body(level, parts_ref, grow_ref, rrow_ref, gout_ref, rout_ref):
    h = _sum_parts(parts_ref)
    gprev = _from_row(grow_ref[...])
    rank = _from_row(rrow_ref[...])
    bucket, resid = _pick_bucket(h, rank)
    if level == 2:
        gout_ref[...] = _to_row(gprev * _NB2 + bucket, -1)
        rout_ref[...] = _to_row(resid, 0)
    else:
        tbits = gprev * _NB3 + bucket
        t = lax.bitcast_convert_type(tbits, jnp.float32)
        gout_ref[...] = _to_row(t, -1.0)
        rout_ref[...] = _to_row(resid, 0)


def _tc_select(level, parts, grow=None, rrow=None):
    outs = [
        jax.ShapeDtypeStruct((1, 128),
                             jnp.float32 if level == 3 else jnp.int32),
        jax.ShapeDtypeStruct((1, 128), jnp.int32),
    ]
    if level == 1:
        return pl.pallas_call(_s1_body, out_shape=outs)(parts)
    body = functools.partial(_s23_body, level)
    return pl.pallas_call(body, out_shape=outs)(parts, grow, rrow)


def _p7_body(nsteps, inv_n, trow_ref, gray_ref, bin_ref, dcp_ref, loss_ref,
             acc_ref):
    i = pl.program_id(0)
    g = gray_ref[...]
    binv = bin_ref[...]
    tpix = jnp.full(g.shape, -1.0, jnp.float32)
    for b in range(_ROWS):
        tpix = jnp.where(binv == b, trow_ref[0, b], tpix)
    dcp = jnp.where(g <= tpix, g, 0.0)
    dcp_ref[...] = dcp

    @pl.when(i == 0)
    def _():
        acc_ref[0] = 0.0

    acc_ref[0] = acc_ref[0] + jnp.sum(jnp.abs(dcp))

    @pl.when(i == nsteps - 1)
    def _():
        loss_ref[...] = jnp.full((8, 128), acc_ref[0] * inv_n, jnp.float32)


def _tc_final(gray2d, bin2d, trow):
    rows, cols = gray2d.shape
    blk = 256
    nsteps = rows // blk
    n = rows * cols
    body = functools.partial(_p7_body, nsteps, 1.0 / n)
    return pl.pallas_call(
        body,
        grid=(nsteps,),
        in_specs=[
            pl.BlockSpec(memory_space=pltpu.SMEM),
            pl.BlockSpec((blk, cols), lambda i: (i, 0)),
            pl.BlockSpec((blk, cols), lambda i: (i, 0)),
        ],
        out_specs=[
            pl.BlockSpec((blk, cols), lambda i: (i, 0)),
            pl.BlockSpec((8, 128), lambda i: (0, 0)),
        ],
        out_shape=[
            jax.ShapeDtypeStruct((rows, cols), jnp.float32),
            jax.ShapeDtypeStruct((8, 128), jnp.float32),
        ],
        scratch_shapes=[pltpu.SMEM((1,), jnp.float32)],
    )(trow, gray2d, bin2d)


def kernel(rgb, d):
    B, C, H, W = rgb.shape
    n = B * H * W

    gray, mn_a, mx_a = _tc_gray_minmax(rgb, d)
    dmm = jnp.stack([mn_a[0, 0], mx_a[0, 0]])
    binb = _tc_bins(d, dmm)

    gray_flat = gray.reshape(n)
    bin_flat = binb.reshape(n)

    parts1 = _sc_hist(gray_flat, bin_flat, 1)
    g1, r1 = _tc_select(1, parts1)
    parts2 = _sc_hist(gray_flat, bin_flat, 2, g1.reshape(128))
    g2, r2 = _tc_select(2, parts2, g1, r1)
    parts3 = _sc_hist(gray_flat, bin_flat, 3, g2.reshape(128))
    trow, _ = _tc_select(3, parts3, g2, r2)

    dcp2d, loss_a = _tc_final(gray_flat.reshape(2048, n // 2048),
                              bin_flat.reshape(2048, n // 2048), trow)
    return (loss_a[0, 0], dcp2d.reshape(B, 1, H, W))


# speedup vs baseline: 171.1636x; 1.5471x over previous
import functools

import jax
import jax.numpy as jnp
from jax import lax
from jax.experimental import pallas as pl
from jax.experimental.pallas import tpu as pltpu
from jax.experimental.pallas import tpu_sc as plsc

_NBINS = 10
_PCT = 0.01
_NB1 = 2048
_NB2 = 2048
_NB3 = 1024
_ROWS = _NBINS + 1


def _p0_body(rgb_ref, d_ref, gray_ref, mn_ref, mx_ref):
    i = pl.program_id(0)
    r = rgb_ref[0, 0]
    g = rgb_ref[0, 1]
    b = rgb_ref[0, 2]
    gray_ref[0, 0] = 0.299 * r + 0.587 * g + 0.114 * b
    dv = d_ref[0, 0]
    mn = jnp.min(dv)
    mx = jnp.max(dv)

    @pl.when(i == 0)
    def _():
        mn_ref[...] = jnp.full((8, 128), jnp.inf, jnp.float32)
        mx_ref[...] = jnp.full((8, 128), -jnp.inf, jnp.float32)

    mn_ref[...] = jnp.minimum(mn_ref[...], mn)
    mx_ref[...] = jnp.maximum(mx_ref[...], mx)


def _tc_gray_minmax(rgb, d):
    B, C, H, W = rgb.shape
    return pl.pallas_call(
        _p0_body,
        grid=(B,),
        in_specs=[
            pl.BlockSpec((1, 3, H, W), lambda i: (i, 0, 0, 0)),
            pl.BlockSpec((1, 1, H, W), lambda i: (i, 0, 0, 0)),
        ],
        out_specs=[
            pl.BlockSpec((1, 1, H, W), lambda i: (i, 0, 0, 0)),
            pl.BlockSpec((8, 128), lambda i: (0, 0)),
            pl.BlockSpec((8, 128), lambda i: (0, 0)),
        ],
        out_shape=[
            jax.ShapeDtypeStruct((B, 1, H, W), jnp.float32),
            jax.ShapeDtypeStruct((8, 128), jnp.float32),
            jax.ShapeDtypeStruct((8, 128), jnp.float32),
        ],
    )(rgb, d)


def _p1_body(dmm_ref, d_ref, bin_ref):
    dmin = dmm_ref[0]
    dmax = dmm_ref[1]
    drange = dmax - dmin
    dv = d_ref[0, 0]
    binv = jnp.zeros(dv.shape, jnp.int32)
    for i in range(1, _NBINS + 1):
        lo = dmin + (jnp.float32(i) * drange) / jnp.float32(_NBINS)
        binv = binv + (dv >= lo).astype(jnp.int32)
    bin_ref[0, 0] = binv


def _tc_bins(d, dmm):
    B, _, H, W = d.shape
    return pl.pallas_call(
        _p1_body,
        grid=(B,),
        in_specs=[
            pl.BlockSpec(memory_space=pltpu.SMEM),
            pl.BlockSpec((1, 1, H, W), lambda i: (i, 0, 0, 0)),
        ],
        out_specs=pl.BlockSpec((1, 1, H, W), lambda i: (i, 0, 0, 0)),
        out_shape=jax.ShapeDtypeStruct((B, 1, H, W), jnp.int32),
    )(dmm, d)


_CHUNK = 8192
_UNROLL = 8


def _sc_hist(gray_flat, bin_flat, level, tbl=None):
    n = gray_flat.shape[0]
    info = plsc.get_sparse_core_info()
    nc, ns = info.num_cores, info.num_subcores
    nw = nc * ns
    ew = n // nw
    nch = ew // _CHUNK
    nb = {1: _NB1, 2: _NB2, 3: _NB3}[level]

    mesh = plsc.VectorSubcoreMesh(
        core_axis_name="c", subcore_axis_name="s",
        num_cores=nc, num_subcores=ns)

    scratch = [
        pltpu.VMEM((_CHUNK,), jnp.float32),
        pltpu.VMEM((_CHUNK,), jnp.float32),
        pltpu.VMEM((_CHUNK,), jnp.int32),
        pltpu.VMEM((_CHUNK,), jnp.int32),
        pltpu.VMEM((_ROWS * nb,), jnp.int32),
        pltpu.SemaphoreType.DMA,
        pltpu.SemaphoreType.DMA,
        pltpu.SemaphoreType.DMA,
        pltpu.SemaphoreType.DMA,
    ]
    if level > 1:
        scratch.append(pltpu.VMEM((16,), jnp.int32))

    def body(*refs):
        if level == 1:
            (gray_hbm, bin_hbm, out_hbm, gbuf0, gbuf1, bbuf0, bbuf1, hist,
             sg0, sg1, sb0, sb1) = refs
            tblv = None
        else:
            (gray_hbm, bin_hbm, tbl_hbm, out_hbm, gbuf0, gbuf1, bbuf0, bbuf1,
             hist, sg0, sg1, sb0, sb1, tblv) = refs
        wid = lax.axis_index("s") * nc + lax.axis_index("c")
        base = wid * ew
        gbufs = (gbuf0, gbuf1)
        bbufs = (bbuf0, bbuf1)
        sems = ((sg0, sb0), (sg1, sb1))

        def zrow(j, _):
            for u in range(_UNROLL):
                hist[pl.ds((j * _UNROLL + u) * 16, 16)] = (
                    jnp.zeros((16,), jnp.int32))
            return 0
        lax.fori_loop(0, (_ROWS * nb) // (16 * _UNROLL), zrow, 0)

        if level > 1:
            pltpu.sync_copy(tbl_hbm.at[pl.ds(0, 16)], tblv)

        ones = jnp.ones((16,), jnp.int32)

        def start(c, slot):
            off = base + c * _CHUNK
            pltpu.async_copy(gray_hbm.at[pl.ds(off, _CHUNK)],
                             gbufs[slot], sems[slot][0])
            pltpu.async_copy(bin_hbm.at[pl.ds(off, _CHUNK)],
                             bbufs[slot], sems[slot][1])

        def wait(slot):
            pltpu.make_async_copy(gray_hbm.at[pl.ds(0, _CHUNK)],
                                  gbufs[slot], sems[slot][0]).wait()
            pltpu.make_async_copy(bin_hbm.at[pl.ds(0, _CHUNK)],
                                  bbufs[slot], sems[slot][1]).wait()

        start(0, 0)
        for c in range(nch):
            slot = c & 1
            if c + 1 < nch:
                start(c + 1, 1 - slot)
            wait(slot)
            gb = gbufs[slot]
            bb = bbufs[slot]

            @plsc.parallel_loop(0, _CHUNK, 16, unroll=_UNROLL)
            def _(off, gb=gb, bb=bb):
                s = pl.ds(off, 16)
                bits = lax.bitcast_convert_type(gb[s], jnp.int32)
                binv = bb[s]
                if level == 1:
                    k = lax.shift_right_logical(bits, 21)
                    plsc.addupdate_scatter(hist, [binv * nb + k], ones)
                elif level == 2:
                    t = plsc.load_gather(tblv, [binv])
                    m = lax.shift_right_logical(bits, 21) == t
                    k = lax.shift_right_logical(bits, 10) & 0x7FF
                    plsc.addupdate_scatter(hist, [binv * nb + k], ones,
                                           mask=m)
                else:
                    t = plsc.load_gather(tblv, [binv])
                    m = lax.shift_right_logical(bits, 10) == t
                    k = bits & 0x3FF
                    plsc.addupdate_scatter(hist, [binv * nb + k], ones,
                                           mask=m)

        for r in range(_ROWS):
            pltpu.sync_copy(hist.at[pl.ds(r * nb, nb)],
                            out_hbm.at[wid * 16 + r])

    kern = pl.kernel(
        body,
        out_type=jax.ShapeDtypeStruct((nw * 16, nb), jnp.int32),
        mesh=mesh,
        scratch_types=scratch,
        compiler_params=pltpu.CompilerParams(needs_layout_passes=False),
    )
    if level == 1:
        return kern(gray_flat, bin_flat)
    return kern(gray_flat, bin_flat, tbl)


def _cumsum_rows(h):
    nb = h.shape[1]
    cum = h
    s = 1
    while s < nb:
        z = jnp.zeros((h.shape[0], s), jnp.int32)
        cum = cum + jnp.concatenate([z, cum[:, : nb - s]], axis=1)
        s *= 2
    return cum


def _pick_bucket(h, rank):
    nb = h.shape[1]
    cum = _cumsum_rows(h)
    bucket = jnp.sum((cum <= rank).astype(jnp.int32), axis=1, keepdims=True)
    bucket = jnp.minimum(bucket, nb - 1)
    col = lax.broadcasted_iota(jnp.int32, h.shape, 1)
    below = jnp.sum(jnp.where(col < bucket, h, 0), axis=1, keepdims=True)
    return bucket, rank - below


def _to_row(v, fill):
    rid = lax.broadcasted_iota(jnp.int32, (_ROWS, 128), 0)
    cid = lax.broadcasted_iota(jnp.int32, (_ROWS, 128), 1)
    mat = jnp.where(rid == cid, jnp.broadcast_to(v, (_ROWS, 128)),
                    jnp.zeros((_ROWS, 128), v.dtype))
    row = jnp.sum(mat, axis=0, keepdims=True)
    c = lax.broadcasted_iota(jnp.int32, (1, 128), 1)
    return jnp.where(c >= _NBINS, jnp.asarray(fill, v.dtype), row)


def _from_row(row):
    rid = lax.broadcasted_iota(jnp.int32, (_ROWS, 128), 0)
    cid = lax.broadcasted_iota(jnp.int32, (_ROWS, 128), 1)
    mat = jnp.where(rid == cid, jnp.broadcast_to(row, (_ROWS, 128)),
                    jnp.zeros((_ROWS, 128), jnp.int32))
    return jnp.sum(mat, axis=1, keepdims=True)


def _sum_parts(parts_ref):
    nworkers = parts_ref.shape[0] // 16
    h = parts_ref[0:_ROWS, :]
    for w in range(1, nworkers):
        h = h + parts_ref[w * 16:w * 16 + _ROWS, :]
    return h


def _s1_body(parts_ref, g1_ref, r1_ref):
    h = _sum_parts(parts_ref)
    num = jnp.sum(h, axis=1, keepdims=True)
    kf = jnp.ceil(num.astype(jnp.float32) * jnp.float32(_PCT))
    k = jnp.maximum(kf.astype(jnp.int32) - 1, 0)
    bucket, resid = _pick_bucket(h, k)
    g1_ref[...] = _to_row(bucket, -1)
    r1_ref[...] = _to_row(resid, 0)


def _s23_body(level, parts_ref, grow_ref, rrow_ref, gout_ref, rout_ref):
    h = _sum_parts(parts_ref)
    gprev = _from_row(grow_ref[...])
    rank = _from_row(rrow_ref[...])
    bucket, resid = _pick_bucket(h, rank)
    if level == 2:
        gout_ref[...] = _to_row(gprev * _NB2 + bucket, -1)
        rout_ref[...] = _to_row(resid, 0)
    else:
        tbits = gprev * _NB3 + bucket
        t = lax.bitcast_convert_type(tbits, jnp.float32)
        gout_ref[...] = _to_row(t, -1.0)
        rout_ref[...] = _to_row(resid, 0)


def _tc_select(level, parts, grow=None, rrow=None):
    outs = [
        jax.ShapeDtypeStruct((1, 128),
                             jnp.float32 if level == 3 else jnp.int32),
        jax.ShapeDtypeStruct((1, 128), jnp.int32),
    ]
    if level == 1:
        return pl.pallas_call(_s1_body, out_shape=outs)(parts)
    body = functools.partial(_s23_body, level)
    return pl.pallas_call(body, out_shape=outs)(parts, grow, rrow)


def _p7_body(nsteps, inv_n, trow_ref, gray_ref, bin_ref, dcp_ref, loss_ref,
             acc_ref):
    i = pl.program_id(0)
    g = gray_ref[...]
    binv = bin_ref[...]
    tpix = jnp.full(g.shape, -1.0, jnp.float32)
    for b in range(_ROWS):
        tpix = jnp.where(binv == b, trow_ref[0, b], tpix)
    dcp = jnp.where(g <= tpix, g, 0.0)
    dcp_ref[...] = dcp

    @pl.when(i == 0)
    def _():
        acc_ref[0] = 0.0

    acc_ref[0] = acc_ref[0] + jnp.sum(jnp.abs(dcp))

    @pl.when(i == nsteps - 1)
    def _():
        loss_ref[...] = jnp.full((8, 128), acc_ref[0] * inv_n, jnp.float32)


def _tc_final(gray2d, bin2d, trow):
    rows, cols = gray2d.shape
    blk = 256
    nsteps = rows // blk
    n = rows * cols
    body = functools.partial(_p7_body, nsteps, 1.0 / n)
    return pl.pallas_call(
        body,
        grid=(nsteps,),
        in_specs=[
            pl.BlockSpec(memory_space=pltpu.SMEM),
            pl.BlockSpec((blk, cols), lambda i: (i, 0)),
            pl.BlockSpec((blk, cols), lambda i: (i, 0)),
        ],
        out_specs=[
            pl.BlockSpec((blk, cols), lambda i: (i, 0)),
            pl.BlockSpec((8, 128), lambda i: (0, 0)),
        ],
        out_shape=[
            jax.ShapeDtypeStruct((rows, cols), jnp.float32),
            jax.ShapeDtypeStruct((8, 128), jnp.float32),
        ],
        scratch_shapes=[pltpu.SMEM((1,), jnp.float32)],
    )(trow, gray2d, bin2d)


def kernel(rgb, d):
    B, C, H, W = rgb.shape
    n = B * H * W

    gray, mn_a, mx_a = _tc_gray_minmax(rgb, d)
    dmm = jnp.stack([mn_a[0, 0], mx_a[0, 0]])
    binb = _tc_bins(d, dmm)

    gray_flat = gray.reshape(n)
    bin_flat = binb.reshape(n)

    parts1 = _sc_hist(gray_flat, bin_flat, 1)
    g1, r1 = _tc_select(1, parts1)
    parts2 = _sc_hist(gray_flat, bin_flat, 2, g1.reshape(128))
    g2, r2 = _tc_select(2, parts2, g1, r1)
    parts3 = _sc_hist(gray_flat, bin_flat, 3, g2.reshape(128))
    trow, _ = _tc_select(3, parts3, g2, r2)

    dcp2d, loss_a = _tc_final(gray_flat.reshape(2048, n // 2048),
                              bin_flat.reshape(2048, n // 2048), trow)
    return (loss_a[0, 0], dcp2d.reshape(B, 1, H, W))


# speedup vs baseline: 209.1694x; 1.2220x over previous
import functools

import jax
import jax.numpy as jnp
from jax import lax
from jax.experimental import pallas as pl
from jax.experimental.pallas import tpu as pltpu
from jax.experimental.pallas import tpu_sc as plsc

_NBINS = 10
_PCT = 0.01
_NB1 = 2048
_NB2 = 2048
_NB3 = 1024
_ROWS = _NBINS + 1


def _p0_body(nsteps, rgb_ref, d_ref, gray_ref, dmm_ref, acc_ref):
    i = pl.program_id(0)
    r = rgb_ref[0, 0]
    g = rgb_ref[0, 1]
    b = rgb_ref[0, 2]
    gray_ref[...] = 0.299 * r + 0.587 * g + 0.114 * b
    dv = d_ref[...]
    mn = jnp.min(dv)
    mx = jnp.max(dv)

    @pl.when(i == 0)
    def _():
        acc_ref[0] = mn
        acc_ref[1] = mx

    acc_ref[0] = jnp.minimum(acc_ref[0], mn)
    acc_ref[1] = jnp.maximum(acc_ref[1], mx)

    @pl.when(i == nsteps - 1)
    def _():
        dmm_ref[0] = acc_ref[0]
        dmm_ref[1] = acc_ref[1]


def _tc_gray_minmax(rgb, d2):
    B, C, H, W = rgb.shape
    return pl.pallas_call(
        functools.partial(_p0_body, B),
        grid=(B,),
        in_specs=[
            pl.BlockSpec((1, 3, H, W), lambda i: (i, 0, 0, 0)),
            pl.BlockSpec((H, W), lambda i: (i, 0)),
        ],
        out_specs=[
            pl.BlockSpec((H, W), lambda i: (i, 0)),
            pl.BlockSpec(memory_space=pltpu.SMEM),
        ],
        out_shape=[
            jax.ShapeDtypeStruct((B * H, W), jnp.float32),
            jax.ShapeDtypeStruct((2,), jnp.float32),
        ],
        scratch_shapes=[pltpu.SMEM((2,), jnp.float32)],
    )(rgb, d2)


def _p1_body(dmm_ref, d_ref, bin_ref):
    dmin = dmm_ref[0]
    dmax = dmm_ref[1]
    drange = dmax - dmin
    dv = d_ref[...]
    binv = jnp.zeros(dv.shape, jnp.int32)
    for i in range(1, _NBINS + 1):
        lo = dmin + (jnp.float32(i) * drange) / jnp.float32(_NBINS)
        binv = binv + (dv >= lo).astype(jnp.int32)
    bin_ref[...] = binv


def _tc_bins(d2, dmm):
    rows, cols = d2.shape
    blk = 512
    return pl.pallas_call(
        _p1_body,
        grid=(rows // blk,),
        in_specs=[
            pl.BlockSpec(memory_space=pltpu.SMEM),
            pl.BlockSpec((blk, cols), lambda i: (i, 0)),
        ],
        out_specs=pl.BlockSpec((blk, cols), lambda i: (i, 0)),
        out_shape=jax.ShapeDtypeStruct((rows, cols), jnp.int32),
    )(dmm, d2)


_CHUNK = 8192
_UNROLL = 8


def _sc_hist(gray2d, bin2d, level, tbl=None):
    n = gray2d.shape[0] * gray2d.shape[1]
    info = plsc.get_sparse_core_info()
    nc, ns = info.num_cores, info.num_subcores
    nw = nc * ns
    ew = n // nw
    nch = ew // _CHUNK
    nb = {1: _NB1, 2: _NB2, 3: _NB3}[level]

    mesh = plsc.VectorSubcoreMesh(
        core_axis_name="c", subcore_axis_name="s",
        num_cores=nc, num_subcores=ns)

    scratch = [
        pltpu.VMEM((_CHUNK,), jnp.float32),
        pltpu.VMEM((_CHUNK,), jnp.float32),
        pltpu.VMEM((_CHUNK,), jnp.int32),
        pltpu.VMEM((_CHUNK,), jnp.int32),
        pltpu.VMEM((_ROWS * nb,), jnp.int32),
        pltpu.SemaphoreType.DMA,
        pltpu.SemaphoreType.DMA,
        pltpu.SemaphoreType.DMA,
        pltpu.SemaphoreType.DMA,
    ]
    if level > 1:
        scratch.append(pltpu.VMEM((16,), jnp.int32))

    def body(*refs):
        if level == 1:
            (gray2_hbm, bin2_hbm, out_hbm, gbuf0, gbuf1, bbuf0, bbuf1, hist,
             sg0, sg1, sb0, sb1) = refs
            tblv = None
        else:
            (gray2_hbm, bin2_hbm, tbl_hbm, out_hbm, gbuf0, gbuf1, bbuf0,
             bbuf1, hist, sg0, sg1, sb0, sb1, tblv) = refs
        wid = lax.axis_index("s") * nc + lax.axis_index("c")
        cols = gray2d.shape[1]
        rpc = _CHUNK // cols
        brow = wid * (ew // cols)
        gbufs = (gbuf0, gbuf1)
        bbufs = (bbuf0, bbuf1)
        sems = ((sg0, sb0), (sg1, sb1))

        def zrow(j, _):
            for u in range(_UNROLL):
                hist[pl.ds((j * _UNROLL + u) * 16, 16)] = (
                    jnp.zeros((16,), jnp.int32))
            return 0
        lax.fori_loop(0, (_ROWS * nb) // (16 * _UNROLL), zrow, 0)

        if level > 1:
            pltpu.sync_copy(tbl_hbm.at[pl.ds(0, 16)], tblv)

        ones = jnp.ones((16,), jnp.int32)

        def start(c, slot):
            row0 = brow + c * rpc
            for r in range(rpc):
                pltpu.async_copy(gray2_hbm.at[row0 + r],
                                 gbufs[slot].at[pl.ds(r * cols, cols)],
                                 sems[slot][0])
                pltpu.async_copy(bin2_hbm.at[row0 + r],
                                 bbufs[slot].at[pl.ds(r * cols, cols)],
                                 sems[slot][1])

        def wait(slot):
            for r in range(rpc):
                pltpu.make_async_copy(gray2_hbm.at[0],
                                      gbufs[slot].at[pl.ds(r * cols, cols)],
                                      sems[slot][0]).wait()
                pltpu.make_async_copy(bin2_hbm.at[0],
                                      bbufs[slot].at[pl.ds(r * cols, cols)],
                                      sems[slot][1]).wait()

        start(0, 0)
        for c in range(nch):
            slot = c & 1
            if c + 1 < nch:
                start(c + 1, 1 - slot)
            wait(slot)
            gb = gbufs[slot]
            bb = bbufs[slot]

            @plsc.parallel_loop(0, _CHUNK, 16, unroll=_UNROLL)
            def _(off, gb=gb, bb=bb):
                s = pl.ds(off, 16)
                bits = lax.bitcast_convert_type(gb[s], jnp.int32)
                binv = bb[s]
                if level == 1:
                    k = lax.shift_right_logical(bits, 21)
                    plsc.addupdate_scatter(hist, [binv * nb + k], ones)
                elif level == 2:
                    t = plsc.load_gather(tblv, [binv])
                    m = lax.shift_right_logical(bits, 21) == t
                    k = lax.shift_right_logical(bits, 10) & 0x7FF
                    plsc.addupdate_scatter(hist, [binv * nb + k], ones,
                                           mask=m)
                else:
                    t = plsc.load_gather(tblv, [binv])
                    m = lax.shift_right_logical(bits, 10) == t
                    k = bits & 0x3FF
                    plsc.addupdate_scatter(hist, [binv * nb + k], ones,
                                           mask=m)

        for r in range(_ROWS):
            pltpu.sync_copy(hist.at[pl.ds(r * nb, nb)],
                            out_hbm.at[wid * 16 + r])

    kern = pl.kernel(
        body,
        out_type=jax.ShapeDtypeStruct((nw * 16, nb), jnp.int32),
        mesh=mesh,
        scratch_types=scratch,
        compiler_params=pltpu.CompilerParams(needs_layout_passes=False),
    )
    if level == 1:
        return kern(gray2d, bin2d)
    return kern(gray2d, bin2d, tbl)


def _cumsum_rows(h):
    nb = h.shape[1]
    cum = h
    s = 1
    while s < nb:
        z = jnp.zeros((h.shape[0], s), jnp.int32)
        cum = cum + jnp.concatenate([z, cum[:, : nb - s]], axis=1)
        s *= 2
    return cum


def _pick_bucket(h, rank):
    nb = h.shape[1]
    cum = _cumsum_rows(h)
    bucket = jnp.sum((cum <= rank).astype(jnp.int32), axis=1, keepdims=True)
    bucket = jnp.minimum(bucket, nb - 1)
    col = lax.broadcasted_iota(jnp.int32, h.shape, 1)
    below = jnp.sum(jnp.where(col < bucket, h, 0), axis=1, keepdims=True)
    return bucket, rank - below


def _to_row(v, fill):
    rid = lax.broadcasted_iota(jnp.int32, (_ROWS, 128), 0)
    cid = lax.broadcasted_iota(jnp.int32, (_ROWS, 128), 1)
    mat = jnp.where(rid == cid, jnp.broadcast_to(v, (_ROWS, 128)),
                    jnp.zeros((_ROWS, 128), v.dtype))
    row = jnp.sum(mat, axis=0, keepdims=True)
    c = lax.broadcasted_iota(jnp.int32, (1, 128), 1)
    return jnp.where(c >= _NBINS, jnp.asarray(fill, v.dtype), row)


def _from_row(row):
    rid = lax.broadcasted_iota(jnp.int32, (_ROWS, 128), 0)
    cid = lax.broadcasted_iota(jnp.int32, (_ROWS, 128), 1)
    mat = jnp.where(rid == cid, jnp.broadcast_to(row, (_ROWS, 128)),
                    jnp.zeros((_ROWS, 128), jnp.int32))
    return jnp.sum(mat, axis=1, keepdims=True)


def _sum_parts(parts_ref):
    nworkers = parts_ref.shape[0] // 16
    h = parts_ref[0:_ROWS, :]
    for w in range(1, nworkers):
        h = h + parts_ref[w * 16:w * 16 + _ROWS, :]
    return h


def _s1_body(parts_ref, g1_ref, r1_ref):
    h = _sum_parts(parts_ref)
    num = jnp.sum(h, axis=1, keepdims=True)
    kf = jnp.ceil(num.astype(jnp.float32) * jnp.float32(_PCT))
    k = jnp.maximum(kf.astype(jnp.int32) - 1, 0)
    bucket, resid = _pick_bucket(h, k)
    g1_ref[...] = _to_row(bucket, -1)
    r1_ref[...] = _to_row(resid, 0)


def _s23_body(level, parts_ref, grow_ref, rrow_ref, gout_ref, rout_ref):
    h = _sum_parts(parts_ref)
    gprev = _from_row(grow_ref[...])
    rank = _from_row(rrow_ref[...])
    bucket, resid = _pick_bucket(h, rank)
    if level == 2:
        gout_ref[...] = _to_row(gprev * _NB2 + bucket, -1)
        rout_ref[...] = _to_row(resid, 0)
    else:
        tbits = gprev * _NB3 + bucket
        t = lax.bitcast_convert_type(tbits, jnp.float32)
        gout_ref[...] = _to_row(t, -1.0)
        rout_ref[...] = _to_row(resid, 0)


def _tc_select(level, parts, grow=None, rrow=None):
    outs = [
        jax.ShapeDtypeStruct((1, 128),
                             jnp.float32 if level == 3 else jnp.int32),
        jax.ShapeDtypeStruct((1, 128), jnp.int32),
    ]
    if level == 1:
        return pl.pallas_call(_s1_body, out_shape=outs)(parts)
    body = functools.partial(_s23_body, level)
    return pl.pallas_call(body, out_shape=outs)(parts, grow, rrow)


def _p7_body(nsteps, inv_n, trow_ref, gray_ref, bin_ref, dcp_ref, loss_ref,
             acc_ref):
    i = pl.program_id(0)
    g = gray_ref[...]
    binv = bin_ref[...]
    tpix = jnp.full(g.shape, -1.0, jnp.float32)
    for b in range(_ROWS):
        tpix = jnp.where(binv == b, trow_ref[0, b], tpix)
    dcp = jnp.where(g <= tpix, g, 0.0)
    dcp_ref[...] = dcp

    @pl.when(i == 0)
    def _():
        acc_ref[0] = 0.0

    acc_ref[0] = acc_ref[0] + jnp.sum(jnp.abs(dcp))

    @pl.when(i == nsteps - 1)
    def _():
        loss_ref[...] = jnp.full((8, 128), acc_ref[0] * inv_n, jnp.float32)


def _tc_final(gray2d, bin2d, trow):
    rows, cols = gray2d.shape
    blk = 512
    nsteps = rows // blk
    n = rows * cols
    body = functools.partial(_p7_body, nsteps, 1.0 / n)
    return pl.pallas_call(
        body,
        grid=(nsteps,),
        in_specs=[
            pl.BlockSpec(memory_space=pltpu.SMEM),
            pl.BlockSpec((blk, cols), lambda i: (i, 0)),
            pl.BlockSpec((blk, cols), lambda i: (i, 0)),
        ],
        out_specs=[
            pl.BlockSpec((blk, cols), lambda i: (i, 0)),
            pl.BlockSpec((8, 128), lambda i: (0, 0)),
        ],
        out_shape=[
            jax.ShapeDtypeStruct((rows, cols), jnp.float32),
            jax.ShapeDtypeStruct((8, 128), jnp.float32),
        ],
        scratch_shapes=[pltpu.SMEM((1,), jnp.float32)],
    )(trow, gray2d, bin2d)


def kernel(rgb, d):
    B, C, H, W = rgb.shape

    d2 = d.reshape(B * H, W)
    gray2d, dmm = _tc_gray_minmax(rgb, d2)
    bin2d = _tc_bins(d2, dmm)

    parts1 = _sc_hist(gray2d, bin2d, 1)
    g1, r1 = _tc_select(1, parts1)
    parts2 = _sc_hist(gray2d, bin2d, 2, g1.reshape(128))
    g2, r2 = _tc_select(2, parts2, g1, r1)
    parts3 = _sc_hist(gray2d, bin2d, 3, g2.reshape(128))
    trow, _ = _tc_select(3, parts3, g2, r2)

    dcp2d, loss_a = _tc_final(gray2d, bin2d, trow)
    return (loss_a[0, 0], dcp2d.reshape(B, 1, H, W))


# speedup vs baseline: 246.9825x; 1.1808x over previous
import functools

import jax
import jax.numpy as jnp
from jax import lax
from jax.experimental import pallas as pl
from jax.experimental.pallas import tpu as pltpu
from jax.experimental.pallas import tpu_sc as plsc

_NBINS = 10
_PCT = 0.01
_NB1 = 2048
_NB2 = 2048
_NB3 = 256
_ROWS = _NBINS + 1


def _p0_body(nsteps, rgb_ref, d_ref, gray_ref, dmm_ref, acc_ref):
    i = pl.program_id(0)
    r = rgb_ref[0, 0]
    g = rgb_ref[0, 1]
    b = rgb_ref[0, 2]
    gray_ref[...] = 0.299 * r + 0.587 * g + 0.114 * b
    dv = d_ref[...]
    mn = jnp.min(dv)
    mx = jnp.max(dv)

    @pl.when(i == 0)
    def _():
        acc_ref[0] = mn
        acc_ref[1] = mx

    acc_ref[0] = jnp.minimum(acc_ref[0], mn)
    acc_ref[1] = jnp.maximum(acc_ref[1], mx)

    @pl.when(i == nsteps - 1)
    def _():
        dmm_ref[0] = acc_ref[0]
        dmm_ref[1] = acc_ref[1]


def _tc_gray_minmax(rgb, d2):
    B, C, H, W = rgb.shape
    return pl.pallas_call(
        functools.partial(_p0_body, B),
        grid=(B,),
        in_specs=[
            pl.BlockSpec((1, 3, H, W), lambda i: (i, 0, 0, 0)),
            pl.BlockSpec((H, W), lambda i: (i, 0)),
        ],
        out_specs=[
            pl.BlockSpec((H, W), lambda i: (i, 0)),
            pl.BlockSpec(memory_space=pltpu.SMEM),
        ],
        out_shape=[
            jax.ShapeDtypeStruct((B * H, W), jnp.float32),
            jax.ShapeDtypeStruct((2,), jnp.float32),
        ],
        scratch_shapes=[pltpu.SMEM((2,), jnp.float32)],
    )(rgb, d2)


def _p1_body(dmm_ref, d_ref, gray_ref, kk_ref):
    dmin = dmm_ref[0]
    dmax = dmm_ref[1]
    drange = dmax - dmin
    dv = d_ref[...]
    binv = jnp.zeros(dv.shape, jnp.int32)
    for i in range(1, _NBINS + 1):
        lo = dmin + (jnp.float32(i) * drange) / jnp.float32(_NBINS)
        binv = binv + (dv >= lo).astype(jnp.int32)
    bits = lax.bitcast_convert_type(gray_ref[...], jnp.int32)
    kk_ref[...] = (binv << 27) | lax.shift_right_logical(bits, 3)


def _tc_pack(d2, gray2d, dmm):
    rows, cols = d2.shape
    blk = 512
    return pl.pallas_call(
        _p1_body,
        grid=(rows // blk,),
        in_specs=[
            pl.BlockSpec(memory_space=pltpu.SMEM),
            pl.BlockSpec((blk, cols), lambda i: (i, 0)),
            pl.BlockSpec((blk, cols), lambda i: (i, 0)),
        ],
        out_specs=pl.BlockSpec((blk, cols), lambda i: (i, 0)),
        out_shape=jax.ShapeDtypeStruct((rows, cols), jnp.int32),
    )(dmm, d2, gray2d)


_CHUNK = 8192
_UNROLL = 8


def _sc_hist(kk2d, gray2d, level, tbl=None):
    n = kk2d.shape[0] * kk2d.shape[1]
    cols = kk2d.shape[1]
    info = plsc.get_sparse_core_info()
    nc, ns = info.num_cores, info.num_subcores
    nw = nc * ns
    ew = n // nw
    nch = ew // _CHUNK
    nb = {1: _NB1, 2: _NB2, 3: _NB3}[level]
    rpc = _CHUNK // cols

    mesh = plsc.VectorSubcoreMesh(
        core_axis_name="c", subcore_axis_name="s",
        num_cores=nc, num_subcores=ns)

    scratch = [
        pltpu.VMEM((rpc, cols), jnp.int32),
        pltpu.VMEM((rpc, cols), jnp.int32),
        pltpu.VMEM((_ROWS * nb,), jnp.int32),
        pltpu.SemaphoreType.DMA,
        pltpu.SemaphoreType.DMA,
    ]
    if level == 3:
        scratch += [
            pltpu.VMEM((rpc, cols), jnp.float32),
            pltpu.VMEM((rpc, cols), jnp.float32),
            pltpu.SemaphoreType.DMA,
            pltpu.SemaphoreType.DMA,
        ]
    if level > 1:
        scratch.append(pltpu.VMEM((16,), jnp.int32))

    def body(*refs):
        gbufs = None
        if level == 1:
            (kk_hbm, out_hbm, kbuf0, kbuf1, hist, sk0, sk1) = refs
            tblv = None
        elif level == 2:
            (kk_hbm, tbl_hbm, out_hbm, kbuf0, kbuf1, hist, sk0, sk1,
             tblv) = refs
        else:
            (kk_hbm, gray_hbm, tbl_hbm, out_hbm, kbuf0, kbuf1, hist,
             sk0, sk1, gbuf0, gbuf1, sg0, sg1, tblv) = refs
            gbufs = (gbuf0, gbuf1)
            gsems = (sg0, sg1)
        wid = lax.axis_index("s") * nc + lax.axis_index("c")
        brow = wid * (ew // cols)
        kbufs = (kbuf0, kbuf1)
        ksems = (sk0, sk1)

        def zrow(j, _):
            for u in range(_UNROLL):
                hist[pl.ds((j * _UNROLL + u) * 16, 16)] = (
                    jnp.zeros((16,), jnp.int32))
            return 0
        lax.fori_loop(0, (_ROWS * nb) // (16 * _UNROLL), zrow, 0)

        if level > 1:
            pltpu.sync_copy(tbl_hbm.at[pl.ds(0, 16)], tblv)

        ones = jnp.ones((16,), jnp.int32)

        def start(c, slot):
            row0 = brow + c * rpc
            pltpu.async_copy(kk_hbm.at[pl.ds(row0, rpc), :],
                             kbufs[slot], ksems[slot])
            if level == 3:
                pltpu.async_copy(gray_hbm.at[pl.ds(row0, rpc), :],
                                 gbufs[slot], gsems[slot])

        def wait(slot):
            pltpu.make_async_copy(kk_hbm.at[pl.ds(0, rpc), :],
                                  kbufs[slot], ksems[slot]).wait()
            if level == 3:
                pltpu.make_async_copy(gray_hbm.at[pl.ds(0, rpc), :],
                                      gbufs[slot], gsems[slot]).wait()

        start(0, 0)
        for c in range(nch):
            slot = c & 1
            if c + 1 < nch:
                start(c + 1, 1 - slot)
            wait(slot)
            kb = kbufs[slot]
            gb = gbufs[slot] if level == 3 else None

            csh = cols.bit_length() - 1

            @plsc.parallel_loop(0, _CHUNK, 16, unroll=_UNROLL)
            def _(off, kb=kb, gb=gb):
                r = lax.shift_right_logical(off, csh)
                s = pl.ds(off & (cols - 1), 16)
                kk = kb[r, s]
                if level == 1:
                    idx = lax.shift_right_logical(kk, 16)
                    plsc.addupdate_scatter(hist, [idx], ones)
                elif level == 2:
                    binv = lax.shift_right_logical(kk, 27)
                    t = plsc.load_gather(tblv, [binv])
                    m = lax.shift_right_logical(kk, 16) == t
                    k2 = lax.shift_right_logical(kk, 5) & 0x7FF
                    plsc.addupdate_scatter(hist, [(binv << 11) + k2], ones,
                                           mask=m)
                else:
                    bits = lax.bitcast_convert_type(gb[r, s], jnp.int32)
                    binv = lax.shift_right_logical(kk, 27)
                    t = plsc.load_gather(tblv, [binv])
                    m = lax.shift_right_logical(bits, 8) == t
                    k3 = bits & 0xFF
                    plsc.addupdate_scatter(hist, [(binv << 8) + k3], ones,
                                           mask=m)

        for r in range(_ROWS):
            pltpu.sync_copy(hist.at[pl.ds(r * nb, nb)],
                            out_hbm.at[wid * 16 + r])

    kern = pl.kernel(
        body,
        out_type=jax.ShapeDtypeStruct((nw * 16, nb), jnp.int32),
        mesh=mesh,
        scratch_types=scratch,
        compiler_params=pltpu.CompilerParams(needs_layout_passes=False),
    )
    if level == 1:
        return kern(kk2d)
    if level == 2:
        return kern(kk2d, tbl)
    return kern(kk2d, gray2d, tbl)


def _cumsum_rows(h):
    nb = h.shape[1]
    cum = h
    s = 1
    while s < nb:
        z = jnp.zeros((h.shape[0], s), jnp.int32)
        cum = cum + jnp.concatenate([z, cum[:, : nb - s]], axis=1)
        s *= 2
    return cum


def _pick_bucket(h, rank):
    nb = h.shape[1]
    cum = _cumsum_rows(h)
    bucket = jnp.sum((cum <= rank).astype(jnp.int32), axis=1, keepdims=True)
    bucket = jnp.minimum(bucket, nb - 1)
    col = lax.broadcasted_iota(jnp.int32, h.shape, 1)
    below = jnp.sum(jnp.where(col < bucket, h, 0), axis=1, keepdims=True)
    return bucket, rank - below


def _to_row(v, fill):
    rid = lax.broadcasted_iota(jnp.int32, (_ROWS, 128), 0)
    cid = lax.broadcasted_iota(jnp.int32, (_ROWS, 128), 1)
    mat = jnp.where(rid == cid, jnp.broadcast_to(v, (_ROWS, 128)),
                    jnp.zeros((_ROWS, 128), v.dtype))
    row = jnp.sum(mat, axis=0, keepdims=True)
    c = lax.broadcasted_iota(jnp.int32, (1, 128), 1)
    return jnp.where(c >= _NBINS, jnp.asarray(fill, v.dtype), row)


def _from_row(row):
    rid = lax.broadcasted_iota(jnp.int32, (_ROWS, 128), 0)
    cid = lax.broadcasted_iota(jnp.int32, (_ROWS, 128), 1)
    mat = jnp.where(rid == cid, jnp.broadcast_to(row, (_ROWS, 128)),
                    jnp.zeros((_ROWS, 128), jnp.int32))
    return jnp.sum(mat, axis=1, keepdims=True)


def _sum_parts(parts_ref):
    nworkers = parts_ref.shape[0] // 16
    h = parts_ref[0:_ROWS, :]
    for w in range(1, nworkers):
        h = h + parts_ref[w * 16:w * 16 + _ROWS, :]
    return h


def _s1_body(parts_ref, g1_ref, r1_ref):
    h = _sum_parts(parts_ref)
    num = jnp.sum(h, axis=1, keepdims=True)
    kf = jnp.ceil(num.astype(jnp.float32) * jnp.float32(_PCT))
    k = jnp.maximum(kf.astype(jnp.int32) - 1, 0)
    bucket, resid = _pick_bucket(h, k)
    binid = lax.broadcasted_iota(jnp.int32, (_ROWS, 1), 0)
    g1_ref[...] = _to_row((binid << 11) | bucket, -1)
    r1_ref[...] = _to_row(resid, 0)


def _s23_body(level, parts_ref, grow_ref, rrow_ref, gout_ref, rout_ref):
    h = _sum_parts(parts_ref)
    gprev = _from_row(grow_ref[...])
    rank = _from_row(rrow_ref[...])
    bucket, resid = _pick_bucket(h, rank)
    if level == 2:
        b1 = gprev & 0x7FF
        gout_ref[...] = _to_row((b1 << 11) | bucket, -1)
        rout_ref[...] = _to_row(resid, 0)
    else:
        tbits = (gprev << 8) | bucket
        t = lax.bitcast_convert_type(tbits, jnp.float32)
        gout_ref[...] = _to_row(t, -1.0)
        rout_ref[...] = _to_row(resid, 0)


def _tc_select(level, parts, grow=None, rrow=None):
    outs = [
        jax.ShapeDtypeStruct((1, 128),
                             jnp.float32 if level == 3 else jnp.int32),
        jax.ShapeDtypeStruct((1, 128), jnp.int32),
    ]
    if level == 1:
        return pl.pallas_call(_s1_body, out_shape=outs)(parts)
    body = functools.partial(_s23_body, level)
    return pl.pallas_call(body, out_shape=outs)(parts, grow, rrow)


def _p7_body(nsteps, inv_n, trow_ref, gray_ref, kk_ref, dcp_ref, loss_ref,
             acc_ref):
    i = pl.program_id(0)
    g = gray_ref[...]
    binv = lax.shift_right_logical(kk_ref[...], 27)
    tpix = jnp.full(g.shape, -1.0, jnp.float32)
    for b in range(_ROWS):
        tpix = jnp.where(binv == b, trow_ref[0, b], tpix)
    dcp = jnp.where(g <= tpix, g, 0.0)
    dcp_ref[...] = dcp

    @pl.when(i == 0)
    def _():
        acc_ref[0] = 0.0

    acc_ref[0] = acc_ref[0] + jnp.sum(jnp.abs(dcp))

    @pl.when(i == nsteps - 1)
    def _():
        loss_ref[...] = jnp.full((8, 128), acc_ref[0] * inv_n, jnp.float32)


def _tc_final(gray2d, kk2d, trow):
    rows, cols = gray2d.shape
    blk = 512
    nsteps = rows // blk
    n = rows * cols
    body = functools.partial(_p7_body, nsteps, 1.0 / n)
    return pl.pallas_call(
        body,
        grid=(nsteps,),
        in_specs=[
            pl.BlockSpec(memory_space=pltpu.SMEM),
            pl.BlockSpec((blk, cols), lambda i: (i, 0)),
            pl.BlockSpec((blk, cols), lambda i: (i, 0)),
        ],
        out_specs=[
            pl.BlockSpec((blk, cols), lambda i: (i, 0)),
            pl.BlockSpec((8, 128), lambda i: (0, 0)),
        ],
        out_shape=[
            jax.ShapeDtypeStruct((rows, cols), jnp.float32),
            jax.ShapeDtypeStruct((8, 128), jnp.float32),
        ],
        scratch_shapes=[pltpu.SMEM((1,), jnp.float32)],
    )(trow, gray2d, kk2d)


def kernel(rgb, d):
    B, C, H, W = rgb.shape

    d2 = d.reshape(B * H, W)
    gray2d, dmm = _tc_gray_minmax(rgb, d2)
    kk2d = _tc_pack(d2, gray2d, dmm)

    parts1 = _sc_hist(kk2d, gray2d, 1)
    g1, r1 = _tc_select(1, parts1)
    parts2 = _sc_hist(kk2d, gray2d, 2, g1.reshape(128))
    g2, r2 = _tc_select(2, parts2, g1, r1)
    parts3 = _sc_hist(kk2d, gray2d, 3, g2.reshape(128))
    trow, _ = _tc_select(3, parts3, g2, r2)

    dcp2d, loss_a = _tc_final(gray2d, kk2d, trow)
    return (loss_a[0, 0], dcp2d.reshape(B, 1, H, W))


# speedup vs baseline: 258.9354x; 1.0484x over previous
import functools

import jax
import jax.numpy as jnp
from jax import lax
from jax.experimental import pallas as pl
from jax.experimental.pallas import tpu as pltpu
from jax.experimental.pallas import tpu_sc as plsc

_NBINS = 10
_PCT = 0.01
_NB1 = 2048
_NB2 = 2048
_NB3 = 256
_ROWS = _NBINS + 1


def _p0_body(nb, rgb_ref, d_ref, gray_ref, kk_ref, acc_ref):
    i = pl.program_id(0)
    dv = d_ref[...]

    @pl.when(i < nb)
    def _():
        mn = jnp.min(dv)
        mx = jnp.max(dv)

        @pl.when(i == 0)
        def _():
            acc_ref[0] = mn
            acc_ref[1] = mx

        acc_ref[0] = jnp.minimum(acc_ref[0], mn)
        acc_ref[1] = jnp.maximum(acc_ref[1], mx)

    @pl.when(i >= nb)
    def _():
        r = rgb_ref[0, 0]
        g = rgb_ref[0, 1]
        b = rgb_ref[0, 2]
        gray = 0.299 * r + 0.587 * g + 0.114 * b
        gray_ref[...] = gray
        dmin = acc_ref[0]
        dmax = acc_ref[1]
        drange = dmax - dmin
        binv = jnp.zeros(dv.shape, jnp.int32)
        for j in range(1, _NBINS + 1):
            lo = dmin + (jnp.float32(j) * drange) / jnp.float32(_NBINS)
            binv = binv + (dv >= lo).astype(jnp.int32)
        bits = lax.bitcast_convert_type(gray, jnp.int32)
        kk_ref[...] = (binv << 27) | lax.shift_right_logical(bits, 3)


def _tc_gray_pack(rgb, d2):
    B, C, H, W = rgb.shape
    return pl.pallas_call(
        functools.partial(_p0_body, B),
        grid=(2 * B,),
        in_specs=[
            pl.BlockSpec((1, 3, H, W),
                         lambda i: (jnp.maximum(i - B, 0), 0, 0, 0)),
            pl.BlockSpec((H, W), lambda i: (lax.rem(i, B), 0)),
        ],
        out_specs=[
            pl.BlockSpec((H, W), lambda i: (jnp.maximum(i - B, 0), 0)),
            pl.BlockSpec((H, W), lambda i: (jnp.maximum(i - B, 0), 0)),
        ],
        out_shape=[
            jax.ShapeDtypeStruct((B * H, W), jnp.float32),
            jax.ShapeDtypeStruct((B * H, W), jnp.int32),
        ],
        scratch_shapes=[pltpu.SMEM((2,), jnp.float32)],
    )(rgb, d2)


_CHUNK = 16384
_UNROLL = 8


def _sc_hist(kk2d, gray2d, level, tbl=None):
    n = kk2d.shape[0] * kk2d.shape[1]
    cols = kk2d.shape[1]
    info = plsc.get_sparse_core_info()
    nc, ns = info.num_cores, info.num_subcores
    nw = nc * ns
    ew = n // nw
    nch = ew // _CHUNK
    nb = {1: _NB1, 2: _NB2, 3: _NB3}[level]
    rpc = _CHUNK // cols

    mesh = plsc.VectorSubcoreMesh(
        core_axis_name="c", subcore_axis_name="s",
        num_cores=nc, num_subcores=ns)

    scratch = [
        pltpu.VMEM((rpc, cols), jnp.int32),
        pltpu.VMEM((rpc, cols), jnp.int32),
        pltpu.VMEM((_ROWS * nb,), jnp.int32),
        pltpu.SemaphoreType.DMA,
        pltpu.SemaphoreType.DMA,
    ]
    if level == 3:
        scratch += [
            pltpu.VMEM((rpc, cols), jnp.float32),
            pltpu.VMEM((rpc, cols), jnp.float32),
            pltpu.SemaphoreType.DMA,
            pltpu.SemaphoreType.DMA,
        ]
    if level > 1:
        scratch.append(pltpu.VMEM((16,), jnp.int32))

    def body(*refs):
        gbufs = None
        if level == 1:
            (kk_hbm, out_hbm, kbuf0, kbuf1, hist, sk0, sk1) = refs
            tblv = None
        elif level == 2:
            (kk_hbm, tbl_hbm, out_hbm, kbuf0, kbuf1, hist, sk0, sk1,
             tblv) = refs
        else:
            (kk_hbm, gray_hbm, tbl_hbm, out_hbm, kbuf0, kbuf1, hist,
             sk0, sk1, gbuf0, gbuf1, sg0, sg1, tblv) = refs
            gbufs = (gbuf0, gbuf1)
            gsems = (sg0, sg1)
        wid = lax.axis_index("s") * nc + lax.axis_index("c")
        brow = wid * (ew // cols)
        kbufs = (kbuf0, kbuf1)
        ksems = (sk0, sk1)

        def zrow(j, _):
            for u in range(_UNROLL):
                hist[pl.ds((j * _UNROLL + u) * 16, 16)] = (
                    jnp.zeros((16,), jnp.int32))
            return 0
        lax.fori_loop(0, (_ROWS * nb) // (16 * _UNROLL), zrow, 0)

        if level > 1:
            pltpu.sync_copy(tbl_hbm.at[pl.ds(0, 16)], tblv)

        ones = jnp.ones((16,), jnp.int32)

        def start(c, slot):
            row0 = brow + c * rpc
            pltpu.async_copy(kk_hbm.at[pl.ds(row0, rpc), :],
                             kbufs[slot], ksems[slot])
            if level == 3:
                pltpu.async_copy(gray_hbm.at[pl.ds(row0, rpc), :],
                                 gbufs[slot], gsems[slot])

        def wait(slot):
            pltpu.make_async_copy(kk_hbm.at[pl.ds(0, rpc), :],
                                  kbufs[slot], ksems[slot]).wait()
            if level == 3:
                pltpu.make_async_copy(gray_hbm.at[pl.ds(0, rpc), :],
                                      gbufs[slot], gsems[slot]).wait()

        start(0, 0)
        for c in range(nch):
            slot = c & 1
            if c + 1 < nch:
                start(c + 1, 1 - slot)
            wait(slot)
            kb = kbufs[slot]
            gb = gbufs[slot] if level == 3 else None

            csh = cols.bit_length() - 1

            @plsc.parallel_loop(0, _CHUNK, 16, unroll=_UNROLL)
            def _(off, kb=kb, gb=gb):
                r = lax.shift_right_logical(off, csh)
                s = pl.ds(off & (cols - 1), 16)
                kk = kb[r, s]
                if level == 1:
                    idx = lax.shift_right_logical(kk, 16)
                    plsc.addupdate_scatter(hist, [idx], ones)
                elif level == 2:
                    binv = lax.shift_right_logical(kk, 27)
                    t = plsc.load_gather(tblv, [binv])
                    m = lax.shift_right_logical(kk, 16) == t
                    k2 = lax.shift_right_logical(kk, 5) & 0x7FF
                    plsc.addupdate_scatter(hist, [(binv << 11) + k2], ones,
                                           mask=m)
                else:
                    bits = lax.bitcast_convert_type(gb[r, s], jnp.int32)
                    binv = lax.shift_right_logical(kk, 27)
                    t = plsc.load_gather(tblv, [binv])
                    m = lax.shift_right_logical(bits, 8) == t
                    k3 = bits & 0xFF
                    plsc.addupdate_scatter(hist, [(binv << 8) + k3], ones,
                                           mask=m)

        for r in range(_ROWS):
            pltpu.sync_copy(hist.at[pl.ds(r * nb, nb)],
                            out_hbm.at[wid * 16 + r])

    kern = pl.kernel(
        body,
        out_type=jax.ShapeDtypeStruct((nw * 16, nb), jnp.int32),
        mesh=mesh,
        scratch_types=scratch,
        compiler_params=pltpu.CompilerParams(needs_layout_passes=False),
    )
    if level == 1:
        return kern(kk2d)
    if level == 2:
        return kern(kk2d, tbl)
    return kern(kk2d, gray2d, tbl)


def _cumsum_rows(h):
    nb = h.shape[1]
    cum = h
    s = 1
    while s < nb:
        z = jnp.zeros((h.shape[0], s), jnp.int32)
        cum = cum + jnp.concatenate([z, cum[:, : nb - s]], axis=1)
        s *= 2
    return cum


def _pick_bucket(h, rank):
    nb = h.shape[1]
    cum = _cumsum_rows(h)
    bucket = jnp.sum((cum <= rank).astype(jnp.int32), axis=1, keepdims=True)
    bucket = jnp.minimum(bucket, nb - 1)
    col = lax.broadcasted_iota(jnp.int32, h.shape, 1)
    below = jnp.sum(jnp.where(col < bucket, h, 0), axis=1, keepdims=True)
    return bucket, rank - below


def _to_row(v, fill):
    rid = lax.broadcasted_iota(jnp.int32, (_ROWS, 128), 0)
    cid = lax.broadcasted_iota(jnp.int32, (_ROWS, 128), 1)
    mat = jnp.where(rid == cid, jnp.broadcast_to(v, (_ROWS, 128)),
                    jnp.zeros((_ROWS, 128), v.dtype))
    row = jnp.sum(mat, axis=0, keepdims=True)
    c = lax.broadcasted_iota(jnp.int32, (1, 128), 1)
    return jnp.where(c >= _NBINS, jnp.asarray(fill, v.dtype), row)


def _from_row(row):
    rid = lax.broadcasted_iota(jnp.int32, (_ROWS, 128), 0)
    cid = lax.broadcasted_iota(jnp.int32, (_ROWS, 128), 1)
    mat = jnp.where(rid == cid, jnp.broadcast_to(row, (_ROWS, 128)),
                    jnp.zeros((_ROWS, 128), jnp.int32))
    return jnp.sum(mat, axis=1, keepdims=True)


def _sum_parts(parts_ref):
    nworkers = parts_ref.shape[0] // 16
    h = parts_ref[0:_ROWS, :]
    for w in range(1, nworkers):
        h = h + parts_ref[w * 16:w * 16 + _ROWS, :]
    return h


def _s1_body(parts_ref, g1_ref, r1_ref):
    h = _sum_parts(parts_ref)
    num = jnp.sum(h, axis=1, keepdims=True)
    kf = jnp.ceil(num.astype(jnp.float32) * jnp.float32(_PCT))
    k = jnp.maximum(kf.astype(jnp.int32) - 1, 0)
    bucket, resid = _pick_bucket(h, k)
    binid = lax.broadcasted_iota(jnp.int32, (_ROWS, 1), 0)
    g1_ref[...] = _to_row((binid << 11) | bucket, -1)
    r1_ref[...] = _to_row(resid, 0)


def _s23_body(level, parts_ref, grow_ref, rrow_ref, gout_ref, rout_ref):
    h = _sum_parts(parts_ref)
    gprev = _from_row(grow_ref[...])
    rank = _from_row(rrow_ref[...])
    bucket, resid = _pick_bucket(h, rank)
    if level == 2:
        b1 = gprev & 0x7FF
        gout_ref[...] = _to_row((b1 << 11) | bucket, -1)
        rout_ref[...] = _to_row(resid, 0)
    else:
        tbits = (gprev << 8) | bucket
        t = lax.bitcast_convert_type(tbits, jnp.float32)
        gout_ref[...] = _to_row(t, -1.0)
        rout_ref[...] = _to_row(resid, 0)


def _tc_select(level, parts, grow=None, rrow=None):
    outs = [
        jax.ShapeDtypeStruct((1, 128),
                             jnp.float32 if level == 3 else jnp.int32),
        jax.ShapeDtypeStruct((1, 128), jnp.int32),
    ]
    if level == 1:
        return pl.pallas_call(_s1_body, out_shape=outs)(parts)
    body = functools.partial(_s23_body, level)
    return pl.pallas_call(body, out_shape=outs)(parts, grow, rrow)


def _p7_body(nsteps, inv_n, parts_ref, grow_ref, rrow_ref, gray_ref, kk_ref,
             dcp_ref, loss_ref, trow_ref, acc_ref):
    i = pl.program_id(0)

    @pl.when(i == 0)
    def _():
        h = _sum_parts(parts_ref)
        gprev = _from_row(grow_ref[...])
        rank = _from_row(rrow_ref[...])
        bucket, _ = _pick_bucket(h, rank)
        tbits = (gprev << 8) | bucket
        t = lax.bitcast_convert_type(tbits, jnp.float32)
        trow_ref[...] = _to_row(t, -1.0)
        acc_ref[0] = 0.0

    trow = trow_ref[...]
    g = gray_ref[...]
    binv = lax.shift_right_logical(kk_ref[...], 27)
    tpix = jnp.full(g.shape, -1.0, jnp.float32)
    for b in range(_ROWS):
        tpix = jnp.where(binv == b, trow[0, b], tpix)
    dcp = jnp.where(g <= tpix, g, 0.0)
    dcp_ref[...] = dcp
    acc_ref[0] = acc_ref[0] + jnp.sum(jnp.abs(dcp))

    @pl.when(i == nsteps - 1)
    def _():
        loss_ref[...] = jnp.full((8, 128), acc_ref[0] * inv_n, jnp.float32)


def _tc_final(gray2d, kk2d, parts3, grow, rrow):
    rows, cols = gray2d.shape
    blk = 512
    nsteps = rows // blk
    n = rows * cols
    body = functools.partial(_p7_body, nsteps, 1.0 / n)
    return pl.pallas_call(
        body,
        grid=(nsteps,),
        in_specs=[
            pl.BlockSpec(parts3.shape, lambda i: (0, 0)),
            pl.BlockSpec((1, 128), lambda i: (0, 0)),
            pl.BlockSpec((1, 128), lambda i: (0, 0)),
            pl.BlockSpec((blk, cols), lambda i: (i, 0)),
            pl.BlockSpec((blk, cols), lambda i: (i, 0)),
        ],
        out_specs=[
            pl.BlockSpec((blk, cols), lambda i: (i, 0)),
            pl.BlockSpec((8, 128), lambda i: (0, 0)),
        ],
        out_shape=[
            jax.ShapeDtypeStruct((rows, cols), jnp.float32),
            jax.ShapeDtypeStruct((8, 128), jnp.float32),
        ],
        scratch_shapes=[
            pltpu.VMEM((1, 128), jnp.float32),
            pltpu.SMEM((1,), jnp.float32),
        ],
    )(parts3, grow, rrow, gray2d, kk2d)


def kernel(rgb, d):
    B, C, H, W = rgb.shape

    d2 = d.reshape(B * H, W)
    gray2d, kk2d = _tc_gray_pack(rgb, d2)

    parts1 = _sc_hist(kk2d, gray2d, 1)
    g1, r1 = _tc_select(1, parts1)
    parts2 = _sc_hist(kk2d, gray2d, 2, g1.reshape(128))
    g2, r2 = _tc_select(2, parts2, g1, r1)
    parts3 = _sc_hist(kk2d, gray2d, 3, g2.reshape(128))

    dcp2d, loss_a = _tc_final(gray2d, kk2d, parts3, g2, r2)
    return (loss_a[0, 0], dcp2d.reshape(B, 1, H, W))


# speedup vs baseline: 270.1997x; 1.0435x over previous
import functools

import jax
import jax.numpy as jnp
from jax import lax
from jax.experimental import pallas as pl
from jax.experimental.pallas import tpu as pltpu
from jax.experimental.pallas import tpu_sc as plsc

_NBINS = 10
_PCT = 0.01
_NB1 = 2048
_NB2 = 2048
_NB3 = 256
_ROWS = _NBINS + 1


def _p0_body(nb, rgb_ref, d_ref, gray_ref, kk_ref, acc_ref):
    i = pl.program_id(0)
    dv = d_ref[...]

    @pl.when(i < nb)
    def _():
        mn = jnp.min(dv)
        mx = jnp.max(dv)

        @pl.when(i == 0)
        def _():
            acc_ref[0] = mn
            acc_ref[1] = mx

        acc_ref[0] = jnp.minimum(acc_ref[0], mn)
        acc_ref[1] = jnp.maximum(acc_ref[1], mx)

    @pl.when(i >= nb)
    def _():
        r = rgb_ref[0, 0]
        g = rgb_ref[0, 1]
        b = rgb_ref[0, 2]
        gray = 0.299 * r + 0.587 * g + 0.114 * b
        gray_ref[...] = gray
        dmin = acc_ref[0]
        dmax = acc_ref[1]
        drange = dmax - dmin
        binv = jnp.zeros(dv.shape, jnp.int32)
        for j in range(1, _NBINS + 1):
            lo = dmin + (jnp.float32(j) * drange) / jnp.float32(_NBINS)
            binv = jnp.where(dv >= lo, binv + 1, binv)
        bits = lax.bitcast_convert_type(gray, jnp.int32)
        kk_ref[...] = (binv << 27) | lax.shift_right_logical(bits, 3)


def _tc_gray_pack(rgb, d2):
    B, C, H, W = rgb.shape
    return pl.pallas_call(
        functools.partial(_p0_body, B),
        grid=(2 * B,),
        in_specs=[
            pl.BlockSpec((1, 3, H, W),
                         lambda i: (jnp.maximum(i - B, 0), 0, 0, 0)),
            pl.BlockSpec((H, W), lambda i: (lax.rem(i, B), 0)),
        ],
        out_specs=[
            pl.BlockSpec((H, W), lambda i: (jnp.maximum(i - B, 0), 0)),
            pl.BlockSpec((H, W), lambda i: (jnp.maximum(i - B, 0), 0)),
        ],
        out_shape=[
            jax.ShapeDtypeStruct((B * H, W), jnp.float32),
            jax.ShapeDtypeStruct((B * H, W), jnp.int32),
        ],
        scratch_shapes=[pltpu.SMEM((2,), jnp.float32)],
    )(rgb, d2)


_CHUNK = 16384
_UNROLL = 8


def _sc_hist(kk2d, gray2d, level, tbl=None):
    n = kk2d.shape[0] * kk2d.shape[1]
    cols = kk2d.shape[1]
    info = plsc.get_sparse_core_info()
    nc, ns = info.num_cores, info.num_subcores
    nw = nc * ns
    ew = n // nw
    nch = ew // _CHUNK
    nb = {1: _NB1, 2: _NB2, 3: _NB3}[level]
    rpc = _CHUNK // cols

    mesh = plsc.VectorSubcoreMesh(
        core_axis_name="c", subcore_axis_name="s",
        num_cores=nc, num_subcores=ns)

    scratch = [
        pltpu.VMEM((rpc, cols), jnp.int32),
        pltpu.VMEM((rpc, cols), jnp.int32),
        pltpu.VMEM((_ROWS * nb,), jnp.int32),
        pltpu.SemaphoreType.DMA,
        pltpu.SemaphoreType.DMA,
    ]
    if level == 3:
        scratch += [
            pltpu.VMEM((rpc, cols), jnp.float32),
            pltpu.VMEM((rpc, cols), jnp.float32),
            pltpu.SemaphoreType.DMA,
            pltpu.SemaphoreType.DMA,
        ]
    if level > 1:
        scratch.append(pltpu.VMEM((16,), jnp.int32))

    def body(*refs):
        gbufs = None
        if level == 1:
            (kk_hbm, out_hbm, kbuf0, kbuf1, hist, sk0, sk1) = refs
            tblv = None
        elif level == 2:
            (kk_hbm, tbl_hbm, out_hbm, kbuf0, kbuf1, hist, sk0, sk1,
             tblv) = refs
        else:
            (kk_hbm, gray_hbm, tbl_hbm, out_hbm, kbuf0, kbuf1, hist,
             sk0, sk1, gbuf0, gbuf1, sg0, sg1, tblv) = refs
            gbufs = (gbuf0, gbuf1)
            gsems = (sg0, sg1)
        wid = lax.axis_index("s") * nc + lax.axis_index("c")
        brow = wid * (ew // cols)
        kbufs = (kbuf0, kbuf1)
        ksems = (sk0, sk1)

        ones = jnp.ones((16,), jnp.int32)

        def start(c, slot):
            row0 = brow + c * rpc
            pltpu.async_copy(kk_hbm.at[pl.ds(row0, rpc), :],
                             kbufs[slot], ksems[slot])
            if level == 3:
                pltpu.async_copy(gray_hbm.at[pl.ds(row0, rpc), :],
                                 gbufs[slot], gsems[slot])

        def wait(slot):
            pltpu.make_async_copy(kk_hbm.at[pl.ds(0, rpc), :],
                                  kbufs[slot], ksems[slot]).wait()
            if level == 3:
                pltpu.make_async_copy(gray_hbm.at[pl.ds(0, rpc), :],
                                      gbufs[slot], gsems[slot]).wait()

        start(0, 0)

        if level > 1:
            pltpu.sync_copy(tbl_hbm.at[pl.ds(0, 16)], tblv)

        def zrow(j, _):
            for u in range(_UNROLL):
                hist[pl.ds((j * _UNROLL + u) * 16, 16)] = (
                    jnp.zeros((16,), jnp.int32))
            return 0
        lax.fori_loop(0, (_ROWS * nb) // (16 * _UNROLL), zrow, 0)

        for c in range(nch):
            slot = c & 1
            if c + 1 < nch:
                start(c + 1, 1 - slot)
            wait(slot)
            kb = kbufs[slot]
            gb = gbufs[slot] if level == 3 else None

            csh = cols.bit_length() - 1

            @plsc.parallel_loop(0, _CHUNK, 16, unroll=_UNROLL)
            def _(off, kb=kb, gb=gb):
                r = lax.shift_right_logical(off, csh)
                s = pl.ds(off & (cols - 1), 16)
                kk = kb[r, s]
                if level == 1:
                    idx = lax.shift_right_logical(kk, 16)
                    plsc.addupdate_scatter(hist, [idx], ones)
                elif level == 2:
                    binv = lax.shift_right_logical(kk, 27)
                    t = plsc.load_gather(tblv, [binv])
                    m = lax.shift_right_logical(kk, 16) == t
                    k2 = lax.shift_right_logical(kk, 5) & 0x7FF
                    plsc.addupdate_scatter(hist, [(binv << 11) + k2], ones,
                                           mask=m)
                else:
                    bits = lax.bitcast_convert_type(gb[r, s], jnp.int32)
                    binv = lax.shift_right_logical(kk, 27)
                    t = plsc.load_gather(tblv, [binv])
                    m = lax.shift_right_logical(bits, 8) == t
                    k3 = bits & 0xFF
                    plsc.addupdate_scatter(hist, [(binv << 8) + k3], ones,
                                           mask=m)

        for r in range(_ROWS):
            pltpu.sync_copy(hist.at[pl.ds(r * nb, nb)],
                            out_hbm.at[wid * 16 + r])

    kern = pl.kernel(
        body,
        out_type=jax.ShapeDtypeStruct((nw * 16, nb), jnp.int32),
        mesh=mesh,
        scratch_types=scratch,
        compiler_params=pltpu.CompilerParams(needs_layout_passes=False),
    )
    if level == 1:
        return kern(kk2d)
    if level == 2:
        return kern(kk2d, tbl)
    return kern(kk2d, gray2d, tbl)


def _cumsum_rows(h):
    nb = h.shape[1]
    cum = h
    s = 1
    while s < nb:
        z = jnp.zeros((h.shape[0], s), jnp.int32)
        cum = cum + jnp.concatenate([z, cum[:, : nb - s]], axis=1)
        s *= 2
    return cum


def _pick_bucket(h, rank):
    nb = h.shape[1]
    cum = _cumsum_rows(h)
    bucket = jnp.sum((cum <= rank).astype(jnp.int32), axis=1, keepdims=True)
    bucket = jnp.minimum(bucket, nb - 1)
    col = lax.broadcasted_iota(jnp.int32, h.shape, 1)
    below = jnp.sum(jnp.where(col < bucket, h, 0), axis=1, keepdims=True)
    return bucket, rank - below


def _to_row(v, fill):
    rid = lax.broadcasted_iota(jnp.int32, (_ROWS, 128), 0)
    cid = lax.broadcasted_iota(jnp.int32, (_ROWS, 128), 1)
    mat = jnp.where(rid == cid, jnp.broadcast_to(v, (_ROWS, 128)),
                    jnp.zeros((_ROWS, 128), v.dtype))
    row = jnp.sum(mat, axis=0, keepdims=True)
    c = lax.broadcasted_iota(jnp.int32, (1, 128), 1)
    return jnp.where(c >= _NBINS, jnp.asarray(fill, v.dtype), row)


def _from_row(row):
    rid = lax.broadcasted_iota(jnp.int32, (_ROWS, 128), 0)
    cid = lax.broadcasted_iota(jnp.int32, (_ROWS, 128), 1)
    mat = jnp.where(rid == cid, jnp.broadcast_to(row, (_ROWS, 128)),
                    jnp.zeros((_ROWS, 128), jnp.int32))
    return jnp.sum(mat, axis=1, keepdims=True)


def _sum_parts(parts_ref):
    nworkers = parts_ref.shape[0] // 16
    h = parts_ref[0:_ROWS, :]
    for w in range(1, nworkers):
        h = h + parts_ref[w * 16:w * 16 + _ROWS, :]
    return h


def _s1_body(parts_ref, g1_ref, r1_ref):
    h = _sum_parts(parts_ref)
    num = jnp.sum(h, axis=1, keepdims=True)
    kf = jnp.ceil(num.astype(jnp.float32) * jnp.float32(_PCT))
    k = jnp.maximum(kf.astype(jnp.int32) - 1, 0)
    bucket, resid = _pick_bucket(h, k)
    binid = lax.broadcasted_iota(jnp.int32, (_ROWS, 1), 0)
    g1_ref[...] = _to_row((binid << 11) | bucket, -1)
    r1_ref[...] = _to_row(resid, 0)


def _s23_body(level, parts_ref, grow_ref, rrow_ref, gout_ref, rout_ref):
    h = _sum_parts(parts_ref)
    gprev = _from_row(grow_ref[...])
    rank = _from_row(rrow_ref[...])
    bucket, resid = _pick_bucket(h, rank)
    if level == 2:
        b1 = gprev & 0x7FF
        gout_ref[...] = _to_row((b1 << 11) | bucket, -1)
        rout_ref[...] = _to_row(resid, 0)
    else:
        tbits = (gprev << 8) | bucket
        t = lax.bitcast_convert_type(tbits, jnp.float32)
        gout_ref[...] = _to_row(t, -1.0)
        rout_ref[...] = _to_row(resid, 0)


def _tc_select(level, parts, grow=None, rrow=None):
    outs = [
        jax.ShapeDtypeStruct((1, 128),
                             jnp.float32 if level == 3 else jnp.int32),
        jax.ShapeDtypeStruct((1, 128), jnp.int32),
    ]
    if level == 1:
        return pl.pallas_call(_s1_body, out_shape=outs)(parts)
    body = functools.partial(_s23_body, level)
    return pl.pallas_call(body, out_shape=outs)(parts, grow, rrow)


def _p7_body(nsteps, inv_n, parts_ref, grow_ref, rrow_ref, gray_ref, kk_ref,
             dcp_ref, loss_ref, trow_ref, acc_ref):
    i = pl.program_id(0)

    @pl.when(i == 0)
    def _():
        h = _sum_parts(parts_ref)
        gprev = _from_row(grow_ref[...])
        rank = _from_row(rrow_ref[...])
        bucket, _ = _pick_bucket(h, rank)
        tbits = (gprev << 8) | bucket
        t = lax.bitcast_convert_type(tbits, jnp.float32)
        trow_ref[...] = _to_row(t, -1.0)
        acc_ref[0] = 0.0

    trow = trow_ref[...]
    g = gray_ref[...]
    binv = lax.shift_right_logical(kk_ref[...], 27)
    tpix = jnp.full(g.shape, -1.0, jnp.float32)
    for b in range(_ROWS):
        tpix = jnp.where(binv == b, trow[0, b], tpix)
    dcp = jnp.where(g <= tpix, g, 0.0)
    dcp_ref[...] = dcp
    acc_ref[0] = acc_ref[0] + jnp.sum(jnp.abs(dcp))

    @pl.when(i == nsteps - 1)
    def _():
        loss_ref[...] = jnp.full((8, 128), acc_ref[0] * inv_n, jnp.float32)


def _tc_final(gray2d, kk2d, parts3, grow, rrow):
    rows, cols = gray2d.shape
    blk = 512
    nsteps = rows // blk
    n = rows * cols
    body = functools.partial(_p7_body, nsteps, 1.0 / n)
    return pl.pallas_call(
        body,
        grid=(nsteps,),
        in_specs=[
            pl.BlockSpec(parts3.shape, lambda i: (0, 0)),
            pl.BlockSpec((1, 128), lambda i: (0, 0)),
            pl.BlockSpec((1, 128), lambda i: (0, 0)),
            pl.BlockSpec((blk, cols), lambda i: (i, 0)),
            pl.BlockSpec((blk, cols), lambda i: (i, 0)),
        ],
        out_specs=[
            pl.BlockSpec((blk, cols), lambda i: (i, 0)),
            pl.BlockSpec((8, 128), lambda i: (0, 0)),
        ],
        out_shape=[
            jax.ShapeDtypeStruct((rows, cols), jnp.float32),
            jax.ShapeDtypeStruct((8, 128), jnp.float32),
        ],
        scratch_shapes=[
            pltpu.VMEM((1, 128), jnp.float32),
            pltpu.SMEM((1,), jnp.float32),
        ],
    )(parts3, grow, rrow, gray2d, kk2d)


def kernel(rgb, d):
    B, C, H, W = rgb.shape

    d2 = d.reshape(B * H, W)
    gray2d, kk2d = _tc_gray_pack(rgb, d2)

    parts1 = _sc_hist(kk2d, gray2d, 1)
    g1, r1 = _tc_select(1, parts1)
    parts2 = _sc_hist(kk2d, gray2d, 2, g1.reshape(128))
    g2, r2 = _tc_select(2, parts2, g1, r1)
    parts3 = _sc_hist(kk2d, gray2d, 3, g2.reshape(128))

    dcp2d, loss_a = _tc_final(gray2d, kk2d, parts3, g2, r2)
    return (loss_a[0, 0], dcp2d.reshape(B, 1, H, W))
